# Initial kernel scaffold; baseline (speedup 1.0000x reference)
import jax
import jax.numpy as jnp
from jax.experimental import pallas as pl


def kernel(x, edge_index, batch, W1, b1, W2, b2, W3, b3, M1, mb1, M2, mb2):
    raise NotImplementedError("write your pallas kernel here")


# speedup vs baseline: 3.3391x; 3.3391x over previous
import functools

import jax
import jax.numpy as jnp
from jax import lax
from jax.experimental import pallas as pl
from jax.experimental.pallas import tpu as pltpu
from jax.experimental.pallas import tpu_sc as plsc

N = 10000
E = 320000
IN_DIM = 128
HID = 256
OUT_DIM = 78
G = 256
HALF = HID // 2
NC, NS = 2, 16
CHUNK = 128
E_PAD = 327680
NCHUNK = E_PAD // CHUNK
AGG_JC = NCHUNK // NS
DEG_JC = NCHUNK // (NC * NS)
NHALF = 5000
NSAC = 64
NH = NHALF + NSAC
SUB5 = 312
PAD_DST = 1 << 20
R = 1000
NBLK = N // R

_f32 = jnp.float32
_HIGH = lax.Precision.HIGHEST


def _clamp_slices(dst_src_ref, dstv_ref, p):
    @pl.loop(0, CHUNK // 16)
    def _(kk):
        sl = pl.ds(kk * 16, 16)
        dv = dst_src_ref[sl]
        sac = NHALF + (dv & (NSAC - 1))
        if p == 0:
            dstv_ref[sl] = jnp.where(dv < NHALF, dv, sac)
        else:
            ok = (dv >= NHALF) & (dv < 2 * NHALF)
            dstv_ref[sl] = jnp.where(ok, dv - NHALF, sac)


def _sc_deg_body(dst_hbm, out_hbm, dstbuf, dstv, ones_v, zbuf, acc_sh):
    c = lax.axis_index("c")
    s = lax.axis_index("s")

    @pl.loop(0, CHUNK)
    def _(i):
        @pl.loop(0, 8)
        def _(kk):
            ones_v[i, pl.ds(kk * 16, 16)] = jnp.full((16,), 1.0, _f32)

    @pl.loop(0, SUB5)
    def _(i):
        @pl.loop(0, 8)
        def _(kk):
            zbuf[i, pl.ds(kk * 16, 16)] = jnp.zeros((16,), _f32)

    base_cid = c * (NCHUNK // NC) + s * DEG_JC
    pltpu.sync_copy(dst_hbm.at[pl.ds(base_cid * CHUNK, DEG_JC * CHUNK)],
                    dstbuf)

    for p in range(2):
        base = s * SUB5
        pltpu.sync_copy(zbuf, acc_sh.at[pl.ds(base, SUB5)])

        @pl.when(s == NS - 1)
        def _():
            pltpu.sync_copy(zbuf.at[pl.ds(0, 8)],
                            acc_sh.at[pl.ds(NS * SUB5, 8)])

        plsc.subcore_barrier()

        @pl.loop(0, DEG_JC)
        def _(j):
            _clamp_slices(dstbuf.at[pl.ds(j * CHUNK, CHUNK)], dstv, p)
            pltpu.sync_copy(ones_v, acc_sh.at[dstv], add=True)

        plsc.subcore_barrier()
        lo = p * NHALF
        pltpu.sync_copy(acc_sh.at[pl.ds(base, SUB5)],
                        out_hbm.at[pl.ds(c * N + lo + base, SUB5), :])

        @pl.when(s == NS - 1)
        def _():
            pltpu.sync_copy(acc_sh.at[pl.ds(NS * SUB5, 8)],
                            out_hbm.at[pl.ds(c * N + lo + NS * SUB5, 8), :])

        plsc.subcore_barrier()


def _sc_agg_body(scaled_hbm, src2_hbm, dst_hbm, out_hbm, srcbuf, dstbuf,
                 dstv, rows_v, acc_sh, sem):
    c = lax.axis_index("c")
    s = lax.axis_index("s")

    base_e = s * AGG_JC * CHUNK
    pltpu.sync_copy(src2_hbm.at[pl.ds(c * E_PAD + base_e, AGG_JC * CHUNK)],
                    srcbuf)
    pltpu.sync_copy(dst_hbm.at[pl.ds(base_e, AGG_JC * CHUNK)], dstbuf)

    for p in range(2):
        lo = p * NHALF
        base = s * SUB5
        pltpu.sync_copy(scaled_hbm.at[pl.ds(c * N + lo + base, SUB5), :],
                        acc_sh.at[pl.ds(base, SUB5)])

        @pl.when(s == NS - 1)
        def _():
            pltpu.sync_copy(
                scaled_hbm.at[pl.ds(c * N + lo + NS * SUB5, 8), :],
                acc_sh.at[pl.ds(NS * SUB5, 8)])

        plsc.subcore_barrier()

        @pl.loop(0, AGG_JC)
        def _(j):
            pltpu.async_copy(
                scaled_hbm.at[srcbuf.at[pl.ds(j * CHUNK, CHUNK)]],
                rows_v, sem).wait()
            _clamp_slices(dstbuf.at[pl.ds(j * CHUNK, CHUNK)], dstv, p)
            pltpu.sync_copy(rows_v, acc_sh.at[dstv], add=True)

        plsc.subcore_barrier()
        pltpu.sync_copy(acc_sh.at[pl.ds(base, SUB5)],
                        out_hbm.at[pl.ds(c * N + lo + base, SUB5), :])

        @pl.when(s == NS - 1)
        def _():
            pltpu.sync_copy(acc_sh.at[pl.ds(NS * SUB5, 8)],
                            out_hbm.at[pl.ds(c * N + lo + NS * SUB5, 8), :])

        plsc.subcore_barrier()


@functools.cache
def _sc_kernels():
    mesh = plsc.VectorSubcoreMesh(core_axis_name="c", subcore_axis_name="s")
    sc_deg = functools.partial(
        pl.kernel,
        mesh=mesh,
        out_type=jax.ShapeDtypeStruct((NC * N, 128), _f32),
        scratch_types=[
            pltpu.VMEM((DEG_JC * CHUNK,), jnp.int32),
            pltpu.VMEM((CHUNK,), jnp.int32),
            pltpu.VMEM((CHUNK, 128), _f32),
            pltpu.VMEM((SUB5, 128), _f32),
            pltpu.VMEM_SHARED((NH, 128), _f32),
        ],
    )(_sc_deg_body)
    sc_agg = functools.partial(
        pl.kernel,
        mesh=mesh,
        out_type=jax.ShapeDtypeStruct((NC * N, 128), _f32),
        scratch_types=[
            pltpu.VMEM((AGG_JC * CHUNK,), jnp.int32),
            pltpu.VMEM((AGG_JC * CHUNK,), jnp.int32),
            pltpu.VMEM((CHUNK,), jnp.int32),
            pltpu.VMEM((CHUNK, 128), _f32),
            pltpu.VMEM_SHARED((NH, 128), _f32),
            pltpu.SemaphoreType.DMA,
        ],
    )(_sc_agg_body)
    return sc_deg, sc_agg


def _sc_deg(dst):
    return _sc_kernels()[0](dst)


def _sc_agg(scaled, src2, dst):
    return _sc_kernels()[1](scaled, src2, dst)


def _dinv_from_parts(dp_ref):
    deg = dp_ref[0, :, 0:1] + dp_ref[1, :, 0:1] + 1.0
    return lax.rsqrt(deg)


def _t1_body(dp_ref, x_ref, w_ref, out_ref):
    dinv = _dinv_from_parts(dp_ref)
    xw = jnp.dot(x_ref[...], w_ref[...], precision=_HIGH,
                 preferred_element_type=_f32)
    scaled = xw * dinv
    out_ref[0] = scaled[:, :HALF]
    out_ref[1] = scaled[:, HALF:]


def _t2_body(acc_ref, dp_ref, b_ref, w_ref, out_ref):
    dinv = _dinv_from_parts(dp_ref)
    acc = jnp.concatenate([acc_ref[0], acc_ref[1]], axis=1)
    h = jnp.maximum(acc * dinv + b_ref[...], 0.0)
    scaled = jnp.dot(h, w_ref[...], precision=_HIGH,
                     preferred_element_type=_f32) * dinv
    out_ref[0] = scaled[:, :HALF]
    out_ref[1] = scaled[:, HALF:]


def _t4_body(acc_ref, dp_ref, b_ref, batch_ref, m1_ref, mb1_ref, m2_ref,
             mb2_ref, out_ref, sums_ref, cnt_ref):
    i = pl.program_id(0)

    @pl.when(i == 0)
    def _():
        sums_ref[...] = jnp.zeros_like(sums_ref)
        cnt_ref[...] = jnp.zeros_like(cnt_ref)

    dinv = _dinv_from_parts(dp_ref)
    acc = jnp.concatenate([acc_ref[0], acc_ref[1]], axis=1)
    h = jnp.maximum(acc * dinv + b_ref[...], 0.0)
    gi = lax.broadcasted_iota(jnp.int32, (R, G), 1)
    onehot = (batch_ref[0] == gi).astype(_f32)
    sums_ref[...] += lax.dot_general(onehot, h, (((0,), (0,)), ((), ())),
                                     precision=_HIGH,
                                     preferred_element_type=_f32)
    cnt_ref[...] += lax.dot_general(onehot, jnp.ones((R, 128), _f32),
                                    (((0,), (0,)), ((), ())),
                                    precision=_HIGH,
                                    preferred_element_type=_f32)

    @pl.when(i == pl.num_programs(0) - 1)
    def _():
        g = sums_ref[...] / jnp.maximum(cnt_ref[:, 0:1], 1.0)
        a1 = jnp.maximum(jnp.dot(g, m1_ref[...], precision=_HIGH,
                                 preferred_element_type=_f32) + mb1_ref[...],
                         0.0)
        logits = jnp.dot(a1, m2_ref[...], precision=_HIGH,
                         preferred_element_type=_f32) + mb2_ref[...]
        out_ref[...] = jax.nn.sigmoid(logits)


def _t1(deg_parts, x, W1):
    return pl.pallas_call(
        _t1_body,
        grid=(NBLK,),
        in_specs=[
            pl.BlockSpec((2, R, 128), lambda i: (0, i, 0)),
            pl.BlockSpec((R, IN_DIM), lambda i: (i, 0)),
            pl.BlockSpec((IN_DIM, HID), lambda i: (0, 0)),
        ],
        out_specs=pl.BlockSpec((2, R, HALF), lambda i: (0, i, 0)),
        out_shape=jax.ShapeDtypeStruct((2, N, HALF), _f32),
    )(deg_parts, x, W1)


def _t2(acc, deg_parts, b_prev, W_next):
    return pl.pallas_call(
        _t2_body,
        grid=(NBLK,),
        in_specs=[
            pl.BlockSpec((2, R, HALF), lambda i: (0, i, 0)),
            pl.BlockSpec((2, R, 128), lambda i: (0, i, 0)),
            pl.BlockSpec((1, HID), lambda i: (0, 0)),
            pl.BlockSpec((HID, HID), lambda i: (0, 0)),
        ],
        out_specs=pl.BlockSpec((2, R, HALF), lambda i: (0, i, 0)),
        out_shape=jax.ShapeDtypeStruct((2, N, HALF), _f32),
    )(acc, deg_parts, b_prev, W_next)


def _t4(acc, deg_parts, b3, batch3, M1, mb1, M2, mb2):
    return pl.pallas_call(
        _t4_body,
        grid=(NBLK,),
        in_specs=[
            pl.BlockSpec((2, R, HALF), lambda i: (0, i, 0)),
            pl.BlockSpec((2, R, 128), lambda i: (0, i, 0)),
            pl.BlockSpec((1, HID), lambda i: (0, 0)),
            pl.BlockSpec((1, R, 1), lambda i: (i, 0, 0)),
            pl.BlockSpec((HID, HID), lambda i: (0, 0)),
            pl.BlockSpec((1, HID), lambda i: (0, 0)),
            pl.BlockSpec((HID, OUT_DIM), lambda i: (0, 0)),
            pl.BlockSpec((1, OUT_DIM), lambda i: (0, 0)),
        ],
        out_specs=pl.BlockSpec((G, OUT_DIM), lambda i: (0, 0)),
        out_shape=jax.ShapeDtypeStruct((G, OUT_DIM), _f32),
        scratch_shapes=[
            pltpu.VMEM((G, HID), _f32),
            pltpu.VMEM((G, 128), _f32),
        ],
    )(acc, deg_parts, b3, batch3, M1, mb1, M2, mb2)


@jax.jit
def kernel(x, edge_index, batch, W1, b1, W2, b2, W3, b3, M1, mb1, M2, mb2):
    src = edge_index[0]
    dst = edge_index[1]
    npad = E_PAD - E
    src_p = jnp.concatenate([src, jnp.zeros((npad,), jnp.int32)])
    dst_p = jnp.concatenate([dst, jnp.full((npad,), PAD_DST, jnp.int32)])
    src2 = jnp.concatenate([src_p, src_p + N])
    batch3 = batch.reshape(NBLK, R, 1)
    b1r, b2r, b3r = b1.reshape(1, HID), b2.reshape(1, HID), b3.reshape(1, HID)
    mb1r, mb2r = mb1.reshape(1, HID), mb2.reshape(1, OUT_DIM)

    deg_parts = _sc_deg(dst_p).reshape(2, N, 128)

    scaled1 = _t1(deg_parts, x, W1).reshape(2 * N, HALF)
    acc1 = _sc_agg(scaled1, src2, dst_p).reshape(2, N, HALF)
    scaled2 = _t2(acc1, deg_parts, b1r, W2).reshape(2 * N, HALF)
    acc2 = _sc_agg(scaled2, src2, dst_p).reshape(2, N, HALF)
    scaled3 = _t2(acc2, deg_parts, b2r, W3).reshape(2 * N, HALF)
    acc3 = _sc_agg(scaled3, src2, dst_p).reshape(2, N, HALF)
    return _t4(acc3, deg_parts, b3r, batch3, M1, mb1r, M2, mb2r)


# speedup vs baseline: 3.7421x; 1.1207x over previous
import functools

import jax
import jax.numpy as jnp
from jax import lax
from jax.experimental import pallas as pl
from jax.experimental.pallas import tpu as pltpu
from jax.experimental.pallas import tpu_sc as plsc

N = 10000
E = 320000
IN_DIM = 128
HID = 256
OUT_DIM = 78
G = 256
HALF = HID // 2
NC, NS = 2, 16
CHUNK = 128
E_PAD = 327680
NCHUNK = E_PAD // CHUNK
AGG_JC = NCHUNK // NS
DEG_JC = NCHUNK // (NC * NS)
NHALF = 5000
NSAC = 64
NH = NHALF + NSAC
SUB5 = 312
PAD_DST = 1 << 20
R = 1000
NBLK = N // R

_f32 = jnp.float32
_HIGH = lax.Precision.HIGHEST


def _clamp_slices(dst_src_ref, dstv_ref, p):
    @pl.loop(0, CHUNK // 16)
    def _(kk):
        sl = pl.ds(kk * 16, 16)
        dv = dst_src_ref[sl]
        sac = NHALF + (dv & (NSAC - 1))
        if p == 0:
            dstv_ref[sl] = jnp.where(dv < NHALF, dv, sac)
        else:
            ok = (dv >= NHALF) & (dv < 2 * NHALF)
            dstv_ref[sl] = jnp.where(ok, dv - NHALF, sac)


def _sc_deg_body(dst_hbm, out_hbm, dstbuf, dstv, ones_v, zbuf, acc_sh):
    c = lax.axis_index("c")
    s = lax.axis_index("s")

    @pl.loop(0, CHUNK)
    def _(i):
        @pl.loop(0, 8)
        def _(kk):
            ones_v[i, pl.ds(kk * 16, 16)] = jnp.full((16,), 1.0, _f32)

    @pl.loop(0, SUB5)
    def _(i):
        @pl.loop(0, 8)
        def _(kk):
            zbuf[i, pl.ds(kk * 16, 16)] = jnp.zeros((16,), _f32)

    base_cid = c * (NCHUNK // NC) + s * DEG_JC
    pltpu.sync_copy(dst_hbm.at[pl.ds(base_cid * CHUNK, DEG_JC * CHUNK)],
                    dstbuf)

    for p in range(2):
        base = s * SUB5
        pltpu.sync_copy(zbuf, acc_sh.at[pl.ds(base, SUB5)])

        @pl.when(s == NS - 1)
        def _():
            pltpu.sync_copy(zbuf.at[pl.ds(0, 8)],
                            acc_sh.at[pl.ds(NS * SUB5, 8)])

        plsc.subcore_barrier()

        @pl.loop(0, DEG_JC)
        def _(j):
            _clamp_slices(dstbuf.at[pl.ds(j * CHUNK, CHUNK)], dstv, p)
            pltpu.sync_copy(ones_v, acc_sh.at[dstv], add=True)

        plsc.subcore_barrier()
        lo = p * NHALF
        pltpu.sync_copy(acc_sh.at[pl.ds(base, SUB5)],
                        out_hbm.at[pl.ds(c * N + lo + base, SUB5), :])

        @pl.when(s == NS - 1)
        def _():
            pltpu.sync_copy(acc_sh.at[pl.ds(NS * SUB5, 8)],
                            out_hbm.at[pl.ds(c * N + lo + NS * SUB5, 8), :])

        plsc.subcore_barrier()


def _sc_agg_body(scaled_hbm, src2_hbm, dst_hbm, out_hbm, srcbuf, dstbuf,
                 dstv0, dstv1, rows_v0, rows_v1, acc_sh, sem0, sem1):
    c = lax.axis_index("c")
    s = lax.axis_index("s")

    base_e = s * AGG_JC * CHUNK
    pltpu.sync_copy(src2_hbm.at[pl.ds(c * E_PAD + base_e, AGG_JC * CHUNK)],
                    srcbuf.at[pl.ds(0, AGG_JC * CHUNK)])
    pltpu.sync_copy(dst_hbm.at[pl.ds(base_e, AGG_JC * CHUNK)], dstbuf)

    @pl.loop(0, CHUNK // 16)
    def _(kk):
        srcbuf[pl.ds(AGG_JC * CHUNK + kk * 16, 16)] = jnp.zeros(
            (16,), jnp.int32)

    def g_start(j, buf, sem):
        pltpu.async_copy(scaled_hbm.at[srcbuf.at[pl.ds(j * CHUNK, CHUNK)]],
                         buf, sem)

    def g_wait(buf, sem):
        pltpu.make_async_copy(scaled_hbm.at[pl.ds(0, CHUNK), :], buf,
                              sem).wait()

    for p in range(2):
        lo = p * NHALF
        base = s * SUB5
        pltpu.sync_copy(scaled_hbm.at[pl.ds(c * N + lo + base, SUB5), :],
                        acc_sh.at[pl.ds(base, SUB5)])

        @pl.when(s == NS - 1)
        def _():
            pltpu.sync_copy(
                scaled_hbm.at[pl.ds(c * N + lo + NS * SUB5, 8), :],
                acc_sh.at[pl.ds(NS * SUB5, 8)])

        plsc.subcore_barrier()

        g_start(0, rows_v0, sem0)

        @pl.loop(0, AGG_JC // 2)
        def _(jj):
            j0 = jj * 2
            g_start(j0 + 1, rows_v1, sem1)
            _clamp_slices(dstbuf.at[pl.ds(j0 * CHUNK, CHUNK)], dstv0, p)
            g_wait(rows_v0, sem0)
            pltpu.sync_copy(rows_v0, acc_sh.at[dstv0], add=True)
            g_start(j0 + 2, rows_v0, sem0)
            _clamp_slices(dstbuf.at[pl.ds(j0 * CHUNK + CHUNK, CHUNK)],
                          dstv1, p)
            g_wait(rows_v1, sem1)
            pltpu.sync_copy(rows_v1, acc_sh.at[dstv1], add=True)

        g_wait(rows_v0, sem0)
        plsc.subcore_barrier()
        pltpu.sync_copy(acc_sh.at[pl.ds(base, SUB5)],
                        out_hbm.at[pl.ds(c * N + lo + base, SUB5), :])

        @pl.when(s == NS - 1)
        def _():
            pltpu.sync_copy(acc_sh.at[pl.ds(NS * SUB5, 8)],
                            out_hbm.at[pl.ds(c * N + lo + NS * SUB5, 8), :])

        plsc.subcore_barrier()


@functools.cache
def _sc_kernels():
    mesh = plsc.VectorSubcoreMesh(core_axis_name="c", subcore_axis_name="s")
    sc_deg = functools.partial(
        pl.kernel,
        mesh=mesh,
        out_type=jax.ShapeDtypeStruct((NC * N, 128), _f32),
        scratch_types=[
            pltpu.VMEM((DEG_JC * CHUNK,), jnp.int32),
            pltpu.VMEM((CHUNK,), jnp.int32),
            pltpu.VMEM((CHUNK, 128), _f32),
            pltpu.VMEM((SUB5, 128), _f32),
            pltpu.VMEM_SHARED((NH, 128), _f32),
        ],
    )(_sc_deg_body)
    sc_agg = functools.partial(
        pl.kernel,
        mesh=mesh,
        out_type=jax.ShapeDtypeStruct((NC * N, 128), _f32),
        scratch_types=[
            pltpu.VMEM(((AGG_JC + 1) * CHUNK,), jnp.int32),
            pltpu.VMEM((AGG_JC * CHUNK,), jnp.int32),
            pltpu.VMEM((CHUNK,), jnp.int32),
            pltpu.VMEM((CHUNK,), jnp.int32),
            pltpu.VMEM((CHUNK, 128), _f32),
            pltpu.VMEM((CHUNK, 128), _f32),
            pltpu.VMEM_SHARED((NH, 128), _f32),
            pltpu.SemaphoreType.DMA,
            pltpu.SemaphoreType.DMA,
        ],
    )(_sc_agg_body)
    return sc_deg, sc_agg


def _sc_deg(dst):
    return _sc_kernels()[0](dst)


def _sc_agg(scaled, src2, dst):
    return _sc_kernels()[1](scaled, src2, dst)


def _dinv_from_parts(dp_ref):
    deg = dp_ref[0, :, 0:1] + dp_ref[1, :, 0:1] + 1.0
    return lax.rsqrt(deg)


def _t1_body(dp_ref, x_ref, w_ref, out_ref):
    dinv = _dinv_from_parts(dp_ref)
    xw = jnp.dot(x_ref[...], w_ref[...], precision=_HIGH,
                 preferred_element_type=_f32)
    scaled = xw * dinv
    out_ref[0] = scaled[:, :HALF]
    out_ref[1] = scaled[:, HALF:]


def _t2_body(acc_ref, dp_ref, b_ref, w_ref, out_ref):
    dinv = _dinv_from_parts(dp_ref)
    acc = jnp.concatenate([acc_ref[0], acc_ref[1]], axis=1)
    h = jnp.maximum(acc * dinv + b_ref[...], 0.0)
    scaled = jnp.dot(h, w_ref[...], precision=_HIGH,
                     preferred_element_type=_f32) * dinv
    out_ref[0] = scaled[:, :HALF]
    out_ref[1] = scaled[:, HALF:]


def _t4_body(acc_ref, dp_ref, b_ref, batch_ref, m1_ref, mb1_ref, m2_ref,
             mb2_ref, out_ref, sums_ref, cnt_ref):
    i = pl.program_id(0)

    @pl.when(i == 0)
    def _():
        sums_ref[...] = jnp.zeros_like(sums_ref)
        cnt_ref[...] = jnp.zeros_like(cnt_ref)

    dinv = _dinv_from_parts(dp_ref)
    acc = jnp.concatenate([acc_ref[0], acc_ref[1]], axis=1)
    h = jnp.maximum(acc * dinv + b_ref[...], 0.0)
    gi = lax.broadcasted_iota(jnp.int32, (R, G), 1)
    onehot = (batch_ref[0] == gi).astype(_f32)
    sums_ref[...] += lax.dot_general(onehot, h, (((0,), (0,)), ((), ())),
                                     precision=_HIGH,
                                     preferred_element_type=_f32)
    cnt_ref[...] += lax.dot_general(onehot, jnp.ones((R, 128), _f32),
                                    (((0,), (0,)), ((), ())),
                                    precision=_HIGH,
                                    preferred_element_type=_f32)

    @pl.when(i == pl.num_programs(0) - 1)
    def _():
        g = sums_ref[...] / jnp.maximum(cnt_ref[:, 0:1], 1.0)
        a1 = jnp.maximum(jnp.dot(g, m1_ref[...], precision=_HIGH,
                                 preferred_element_type=_f32) + mb1_ref[...],
                         0.0)
        logits = jnp.dot(a1, m2_ref[...], precision=_HIGH,
                         preferred_element_type=_f32) + mb2_ref[...]
        out_ref[...] = jax.nn.sigmoid(logits)


def _t1(deg_parts, x, W1):
    return pl.pallas_call(
        _t1_body,
        grid=(NBLK,),
        in_specs=[
            pl.BlockSpec((2, R, 128), lambda i: (0, i, 0)),
            pl.BlockSpec((R, IN_DIM), lambda i: (i, 0)),
            pl.BlockSpec((IN_DIM, HID), lambda i: (0, 0)),
        ],
        out_specs=pl.BlockSpec((2, R, HALF), lambda i: (0, i, 0)),
        out_shape=jax.ShapeDtypeStruct((2, N, HALF), _f32),
    )(deg_parts, x, W1)


def _t2(acc, deg_parts, b_prev, W_next):
    return pl.pallas_call(
        _t2_body,
        grid=(NBLK,),
        in_specs=[
            pl.BlockSpec((2, R, HALF), lambda i: (0, i, 0)),
            pl.BlockSpec((2, R, 128), lambda i: (0, i, 0)),
            pl.BlockSpec((1, HID), lambda i: (0, 0)),
            pl.BlockSpec((HID, HID), lambda i: (0, 0)),
        ],
        out_specs=pl.BlockSpec((2, R, HALF), lambda i: (0, i, 0)),
        out_shape=jax.ShapeDtypeStruct((2, N, HALF), _f32),
    )(acc, deg_parts, b_prev, W_next)


def _t4(acc, deg_parts, b3, batch3, M1, mb1, M2, mb2):
    return pl.pallas_call(
        _t4_body,
        grid=(NBLK,),
        in_specs=[
            pl.BlockSpec((2, R, HALF), lambda i: (0, i, 0)),
            pl.BlockSpec((2, R, 128), lambda i: (0, i, 0)),
            pl.BlockSpec((1, HID), lambda i: (0, 0)),
            pl.BlockSpec((1, R, 1), lambda i: (i, 0, 0)),
            pl.BlockSpec((HID, HID), lambda i: (0, 0)),
            pl.BlockSpec((1, HID), lambda i: (0, 0)),
            pl.BlockSpec((HID, OUT_DIM), lambda i: (0, 0)),
            pl.BlockSpec((1, OUT_DIM), lambda i: (0, 0)),
        ],
        out_specs=pl.BlockSpec((G, OUT_DIM), lambda i: (0, 0)),
        out_shape=jax.ShapeDtypeStruct((G, OUT_DIM), _f32),
        scratch_shapes=[
            pltpu.VMEM((G, HID), _f32),
            pltpu.VMEM((G, 128), _f32),
        ],
    )(acc, deg_parts, b3, batch3, M1, mb1, M2, mb2)


@jax.jit
def kernel(x, edge_index, batch, W1, b1, W2, b2, W3, b3, M1, mb1, M2, mb2):
    src = edge_index[0]
    dst = edge_index[1]
    npad = E_PAD - E
    src_p = jnp.concatenate([src, jnp.zeros((npad,), jnp.int32)])
    dst_p = jnp.concatenate([dst, jnp.full((npad,), PAD_DST, jnp.int32)])
    src2 = jnp.concatenate([src_p, src_p + N])
    batch3 = batch.reshape(NBLK, R, 1)
    b1r, b2r, b3r = b1.reshape(1, HID), b2.reshape(1, HID), b3.reshape(1, HID)
    mb1r, mb2r = mb1.reshape(1, HID), mb2.reshape(1, OUT_DIM)

    deg_parts = _sc_deg(dst_p).reshape(2, N, 128)

    scaled1 = _t1(deg_parts, x, W1).reshape(2 * N, HALF)
    acc1 = _sc_agg(scaled1, src2, dst_p).reshape(2, N, HALF)
    scaled2 = _t2(acc1, deg_parts, b1r, W2).reshape(2 * N, HALF)
    acc2 = _sc_agg(scaled2, src2, dst_p).reshape(2, N, HALF)
    scaled3 = _t2(acc2, deg_parts, b2r, W3).reshape(2 * N, HALF)
    acc3 = _sc_agg(scaled3, src2, dst_p).reshape(2, N, HALF)
    return _t4(acc3, deg_parts, b3r, batch3, M1, mb1r, M2, mb2r)
